# baseline (speedup 1.0000x reference)
import jax
import jax.numpy as jnp
from jax.experimental import pallas as pl

NUM_USERS = 25000
NUM_ITEMS = 25000
N_NODES = NUM_USERS + NUM_ITEMS
EMBED_K = 64
HYPER_NUM = 4
TAU = 0.2
ALPHA = 0.2


def _mm_body(x_ref, w_ref, o_ref):
    o_ref[...] = jnp.dot(x_ref[...], w_ref[...], preferred_element_type=jnp.float32)


def _matmul(x, w, bm=1000):
    M, K = x.shape
    N = w.shape[1]
    return pl.pallas_call(
        _mm_body,
        grid=(M // bm,),
        in_specs=[
            pl.BlockSpec((bm, K), lambda i: (i, 0)),
            pl.BlockSpec((K, N), lambda i: (0, 0)),
        ],
        out_specs=pl.BlockSpec((bm, N), lambda i: (i, 0)),
        out_shape=jax.ShapeDtypeStruct((M, N), jnp.float32),
    )(x, w)


def _l2norm(x, eps=1e-12):
    n = jnp.linalg.norm(x, axis=1, keepdims=True)
    return x / jnp.maximum(n, eps)


def _spmm(index, values, x, n_rows):
    return jnp.zeros((n_rows, x.shape[1]), x.dtype).at[index[0]].add(values[:, None] * x[index[1]])


def _gumbel_softmax(logits, key, tau):
    g = jax.random.gumbel(key, logits.shape, logits.dtype)
    return jax.nn.softmax((logits + g) / tau, axis=1)


def kernel(gu, gi, feat_visual, feat_textual, trs_visual, trs_textual, hyper_visual, hyper_textual, r_index, r_values, adj_index, adj_weight, num_inters_inv):
    feats = [feat_visual, feat_textual]
    gkey = jax.random.key(7)

    ihf = []
    for m, (f, t, h) in enumerate([(feat_visual, trs_visual, hyper_visual),
                                   (feat_textual, trs_textual, hyper_textual)]):
        w = jnp.concatenate([t, h], axis=1)
        w = jnp.pad(w, ((0, 0), (0, 128 - w.shape[1])))
        ihf.append(_matmul(f, w))
    item_feats = [ihf[0][:, :EMBED_K], ihf[1][:, :EMBED_K]]
    ih_raw = [ihf[0][:, EMBED_K:EMBED_K + HYPER_NUM], ihf[1][:, EMBED_K:EMBED_K + HYPER_NUM]]

    i_hyper, u_hyper = [], []
    for m in range(2):
        ih = ih_raw[m]
        uh = _spmm(r_index, r_values, ih, NUM_USERS)
        i_hyper.append(_gumbel_softmax(ih, jax.random.fold_in(gkey, 2 * m), TAU))
        u_hyper.append(_gumbel_softmax(uh, jax.random.fold_in(gkey, 2 * m + 1), TAU))

    ego = jnp.concatenate([gu, gi], axis=0)
    layers = [ego]
    e = ego
    for _ in range(2):
        e = _spmm(adj_index, adj_weight, e, N_NODES)
        layers.append(e)
    cge = (layers[0] + layers[1] + layers[2]) / 3.0

    mge_embs = 0.0
    for m in range(2):
        user_feats = _spmm(r_index, r_values, item_feats[m], NUM_USERS) * num_inters_inv[:NUM_USERS, None]
        mge = jnp.concatenate([user_feats, item_feats[m]], axis=0)
        mge = _spmm(adj_index, adj_weight, mge, N_NODES)
        mge_embs = mge_embs + _l2norm(mge)
    lge = cge + mge_embs

    item_cge = cge[NUM_USERS:]
    hyper_outs = []
    ghe = 0.0
    for m in range(2):
        lat = i_hyper[m].T @ item_cge
        i_ret = i_hyper[m] @ lat
        u_ret = u_hyper[m] @ lat
        hyper_outs += [u_ret, i_ret]
        ghe = ghe + jnp.concatenate([u_ret, i_ret], axis=0)
    all_embs = lge + ALPHA * _l2norm(ghe)
    return (all_embs[:NUM_USERS], all_embs[NUM_USERS:],
            hyper_outs[0], hyper_outs[1], hyper_outs[2], hyper_outs[3])


# speedup vs baseline: 5.1057x; 5.1057x over previous
import jax
import jax.numpy as jnp
from jax import lax
from jax.experimental import pallas as pl
from jax.experimental.pallas import tpu as pltpu
from jax.experimental.pallas import tpu_sc as plsc

NUM_USERS = 25000
NUM_ITEMS = 25000
N_NODES = NUM_USERS + NUM_ITEMS
EMBED_K = 64
HYPER_NUM = 4
TAU = 0.2
ALPHA = 0.2

A_ROWS = 50176
R_ROWS = 25088
A_EPAD = 1007616
R_EPAD = 507904


def _sc_spmm(xs, dst2, src2, w, *, n_rows_pad, chunk_core, x_slot, cw, zr,
             b_edges, edge_split=False):
    n_x = len(xs)
    n_chunks = len(chunk_core)
    weighted = w is not None
    br = b_edges // 128
    blocks_total = (dst2.shape[0] * 128) // b_edges
    bpt = blocks_total // (32 if edge_split else 16)
    rpt = n_rows_pad // 16

    mesh = plsc.VectorSubcoreMesh(core_axis_name="c", subcore_axis_name="s")
    out_type = [jax.ShapeDtypeStruct((n_rows_pad, cw), jnp.float32)
                for _ in range(n_chunks)]
    scratch = [pltpu.VMEM((br, 128), jnp.int32), pltpu.VMEM((br, 128), jnp.int32)]
    if weighted:
        scratch.append(pltpu.VMEM((b_edges,), jnp.float32))
    scratch += [
        pltpu.VMEM((b_edges, cw), jnp.float32),
        pltpu.VMEM((zr, cw), jnp.float32),
        pltpu.VMEM_SHARED((n_rows_pad, cw), jnp.float32),
        pltpu.SemaphoreType.DMA,
    ]

    def body(*refs):
        i = 0
        xs_r = refs[i:i + n_x]; i += n_x
        dst_r = refs[i]; src_r = refs[i + 1]; i += 2
        if weighted:
            w_r = refs[i]; i += 1
        outs_r = refs[i:i + n_chunks]; i += n_chunks
        dst_v = refs[i]; src_v = refs[i + 1]; i += 2
        if weighted:
            w_v = refs[i]; i += 1
        rows_v = refs[i]; zbuf = refs[i + 1]; acc = refs[i + 2]; sem = refs[i + 3]

        cid = lax.axis_index("c")
        sid = lax.axis_index("s")
        r0 = pl.multiple_of(sid * rpt, 8)

        def zrow(j, c):
            for c2 in range(cw // 16):
                zbuf[j, pl.ds(c2 * 16, 16)] = jnp.zeros((16,), jnp.float32)
            return c
        lax.fori_loop(0, zr, zrow, 0)

        for ci in range(n_chunks):
            core = chunk_core[ci]
            x_r = xs_r[x_slot[ci]]
            out_r = outs_r[ci]

            def chunk_body(x_r=x_r, out_r=out_r):
                def zc(j, c):
                    pltpu.sync_copy(zbuf, acc.at[pl.ds(r0 + j * zr, zr)])
                    return c
                lax.fori_loop(0, rpt // zr, zc, 0)
                plsc.subcore_barrier()

                if edge_split:
                    base_b = cid * (blocks_total // 2) + sid * bpt
                else:
                    base_b = sid * bpt

                def blk(b, c):
                    bb = (base_b + b) * br
                    pltpu.sync_copy(dst_r.at[pl.ds(bb, br)], dst_v)
                    pltpu.sync_copy(src_r.at[pl.ds(bb, br)], src_v)
                    if weighted:
                        pltpu.sync_copy(w_r.at[pl.ds(bb * 128, b_edges)], w_v)
                    hs = [pltpu.async_copy(x_r.at[src_v.at[j]],
                                           rows_v.at[pl.ds(j * 128, 128)], sem)
                          for j in range(br)]
                    for h in hs:
                        h.wait()
                    if weighted:
                        def emul(g, c2_):
                            e0 = g * 16
                            w16 = w_v[pl.ds(e0, 16)]
                            for k in range(16):
                                wv = w16.at[jnp.full((16,), k, jnp.int32)].get(
                                    mode="promise_in_bounds")
                                for c2 in range(cw // 16):
                                    sl = pl.ds(c2 * 16, 16)
                                    rows_v[e0 + k, sl] = rows_v[e0 + k, sl] * wv
                            return c2_
                        lax.fori_loop(0, b_edges // 16, emul, 0)
                    for j in range(br):
                        pltpu.sync_copy(rows_v.at[pl.ds(j * 128, 128)],
                                        acc.at[dst_v.at[j]], add=True)
                    return c
                lax.fori_loop(0, bpt, blk, 0)
                plsc.subcore_barrier()

                def fl(j, c):
                    pltpu.sync_copy(acc.at[pl.ds(r0 + j * zr, zr)],
                                    out_r.at[pl.ds(r0 + j * zr, zr)])
                    return c
                lax.fori_loop(0, rpt // zr, fl, 0)
                plsc.subcore_barrier()

            pl.when(cid == core)(chunk_body)

    f = pl.kernel(body, out_type=out_type, mesh=mesh, scratch_types=scratch,
                  compiler_params=pltpu.CompilerParams(use_tc_tiling_on_sc=False))
    args = list(xs) + [dst2, src2] + ([w] if weighted else [])
    res = f(*args)
    return list(res) if isinstance(res, (list, tuple)) else [res]


def _pad_edges(dst, src, w, e_pad, pad_dst):
    e = dst.shape[0]
    dst = jnp.concatenate([dst, jnp.full((e_pad - e,), pad_dst, jnp.int32)])
    src = jnp.concatenate([src, jnp.zeros((e_pad - e,), jnp.int32)])
    dst2 = dst.reshape(e_pad // 128, 128)
    src2 = src.reshape(e_pad // 128, 128)
    if w is not None:
        w = jnp.concatenate([w, jnp.zeros((e_pad - e,), jnp.float32)])
    return dst2, src2, w


def _mm_body(x_ref, w_ref, o_ref):
    o_ref[...] = jnp.dot(x_ref[...], w_ref[...], preferred_element_type=jnp.float32)


def _matmul(x, w, bm=1000):
    M, K = x.shape
    N = w.shape[1]
    return pl.pallas_call(
        _mm_body,
        grid=(M // bm,),
        in_specs=[
            pl.BlockSpec((bm, K), lambda i: (i, 0)),
            pl.BlockSpec((K, N), lambda i: (0, 0)),
        ],
        out_specs=pl.BlockSpec((bm, N), lambda i: (i, 0)),
        out_shape=jax.ShapeDtypeStruct((M, N), jnp.float32),
    )(x, w)


def _l2norm(x, eps=1e-12):
    n = jnp.linalg.norm(x, axis=1, keepdims=True)
    return x / jnp.maximum(n, eps)


def _gumbel_softmax(logits, key, tau):
    g = jax.random.gumbel(key, logits.shape, logits.dtype)
    return jax.nn.softmax((logits + g) / tau, axis=1)


def kernel(gu, gi, feat_visual, feat_textual, trs_visual, trs_textual, hyper_visual, hyper_textual, r_index, r_values, adj_index, adj_weight, num_inters_inv):
    gkey = jax.random.key(7)

    ihf = []
    for f, t, h in [(feat_visual, trs_visual, hyper_visual),
                    (feat_textual, trs_textual, hyper_textual)]:
        w = jnp.concatenate([t, h], axis=1)
        w = jnp.pad(w, ((0, 0), (0, 128 - w.shape[1])))
        ihf.append(_matmul(f, w))
    item_feats = [ihf[0][:, :EMBED_K], ihf[1][:, :EMBED_K]]
    ih_raw = [ihf[0][:, EMBED_K:EMBED_K + HYPER_NUM], ihf[1][:, EMBED_K:EMBED_K + HYPER_NUM]]

    rdst2, rsrc2, _ = _pad_edges(r_index[0], r_index[1], None, R_EPAD, R_ROWS - 1)

    ih2 = jnp.concatenate([ih_raw[0], ih_raw[1],
                           jnp.zeros((NUM_ITEMS, 8), jnp.float32)], axis=1)
    uh_parts = _sc_spmm([ih2], rdst2, rsrc2, None, n_rows_pad=R_ROWS,
                        chunk_core=[0, 1], x_slot=[0, 0], cw=16, zr=112,
                        b_edges=512, edge_split=True)
    uh_raw = (uh_parts[0] + uh_parts[1])[:NUM_USERS]

    uf_raw = _sc_spmm(item_feats, rdst2, rsrc2, None, n_rows_pad=R_ROWS,
                      chunk_core=[0, 1], x_slot=[0, 1], cw=64, zr=112,
                      b_edges=256)

    i_hyper, u_hyper = [], []
    for m in range(2):
        i_hyper.append(_gumbel_softmax(ih_raw[m], jax.random.fold_in(gkey, 2 * m), TAU))
        u_hyper.append(_gumbel_softmax(uh_raw[:, 4 * m:4 * m + 4],
                                       jax.random.fold_in(gkey, 2 * m + 1), TAU))

    user_feats = [uf_raw[m][:NUM_USERS] * num_inters_inv[:NUM_USERS, None]
                  for m in range(2)]

    ego = jnp.concatenate([gu, gi], axis=0)
    mge_in = [jnp.concatenate([user_feats[m], item_feats[m]], axis=0) for m in range(2)]

    adst2, asrc2, aw = _pad_edges(adj_index[0], adj_index[1], adj_weight,
                                  A_EPAD, 0)
    x192 = [ego[:, :32], ego[:, 32:],
            mge_in[0][:, :32], mge_in[0][:, 32:],
            mge_in[1][:, :32], mge_in[1][:, 32:]]
    y = _sc_spmm(x192, adst2, asrc2, aw, n_rows_pad=A_ROWS,
                 chunk_core=[0, 1, 0, 1, 0, 1], x_slot=[0, 1, 2, 3, 4, 5],
                 cw=32, zr=112, b_edges=512)
    y = [v[:N_NODES] for v in y]
    layer1 = jnp.concatenate([y[0], y[1]], axis=1)
    mge_out = [jnp.concatenate([y[2], y[3]], axis=1),
               jnp.concatenate([y[4], y[5]], axis=1)]
    l2 = _sc_spmm([y[0], y[1]], adst2, asrc2, aw, n_rows_pad=A_ROWS,
                  chunk_core=[0, 1], x_slot=[0, 1], cw=32, zr=112,
                  b_edges=512)
    layer2 = jnp.concatenate([v[:N_NODES] for v in l2], axis=1)

    cge = (ego + layer1 + layer2) / 3.0
    lge = cge + _l2norm(mge_out[0]) + _l2norm(mge_out[1])

    item_cge = cge[NUM_USERS:]
    hyper_outs = []
    ghe = 0.0
    for m in range(2):
        lat = i_hyper[m].T @ item_cge
        i_ret = i_hyper[m] @ lat
        u_ret = u_hyper[m] @ lat
        hyper_outs += [u_ret, i_ret]
        ghe = ghe + jnp.concatenate([u_ret, i_ret], axis=0)
    all_embs = lge + ALPHA * _l2norm(ghe)
    return (all_embs[:NUM_USERS], all_embs[NUM_USERS:],
            hyper_outs[0], hyper_outs[1], hyper_outs[2], hyper_outs[3])


# speedup vs baseline: 5.5158x; 1.0803x over previous
import jax
import jax.numpy as jnp
from jax import lax
from jax.experimental import pallas as pl
from jax.experimental.pallas import tpu as pltpu
from jax.experimental.pallas import tpu_sc as plsc

NUM_USERS = 25000
NUM_ITEMS = 25000
N_NODES = NUM_USERS + NUM_ITEMS
EMBED_K = 64
HYPER_NUM = 4
TAU = 0.2
ALPHA = 0.2

A_ROWS = 50176
R_ROWS = 25088
A_EPAD = 1007616
R_EPAD = 507904


def _sc_spmm(xs, ed, *, n_rows_pad, chunk_core, x_slot, cw, zr,
             b_edges, weighted, edge_split=False):
    n_x = len(xs)
    n_chunks = len(chunk_core)
    w2 = None
    if weighted:
        ed, w2 = ed
    br = b_edges // 128
    blocks_total = ed.shape[0] // br
    bpt = blocks_total // (32 if edge_split else 16)
    assert bpt % 2 == 0, bpt
    rpt = n_rows_pad // 16

    mesh = plsc.VectorSubcoreMesh(core_axis_name="c", subcore_axis_name="s")
    out_type = [jax.ShapeDtypeStruct((n_rows_pad, cw), jnp.float32)
                for _ in range(n_chunks)]
    scratch = [
        pltpu.VMEM((2, br, 2, 128), jnp.int32),
        pltpu.VMEM((2, br, 128), jnp.float32),
        pltpu.VMEM((2, b_edges, cw), jnp.float32),
        pltpu.VMEM((zr, cw), jnp.float32),
        pltpu.VMEM_SHARED((n_rows_pad, cw), jnp.float32),
        pltpu.SemaphoreType.DMA,
        pltpu.SemaphoreType.DMA,
    ]

    def body(*refs):
        i = 0
        xs_r = refs[i:i + n_x]; i += n_x
        ed_r = refs[i]; i += 1
        if weighted:
            w_r = refs[i]; i += 1
        outs_r = refs[i:i + n_chunks]; i += n_chunks
        ed_v, w_v, rows_v, zbuf, acc, sem_g, sem_s = refs[i:i + 7]

        cid = lax.axis_index("c")
        sid = lax.axis_index("s")
        r0 = pl.multiple_of(sid * rpt, 8)

        def zrow(j, c):
            for c2 in range(cw // 16):
                zbuf[j, pl.ds(c2 * 16, 16)] = jnp.zeros((16,), jnp.float32)
            return c
        lax.fori_loop(0, zr, zrow, 0)

        for ci in range(n_chunks):
            core = chunk_core[ci]
            x_r = xs_r[x_slot[ci]]
            out_r = outs_r[ci]

            def chunk_body(x_r=x_r, out_r=out_r):
                def zc(j, c):
                    pltpu.sync_copy(zbuf, acc.at[pl.ds(r0 + j * zr, zr)])
                    return c
                lax.fori_loop(0, rpt // zr, zc, 0)
                plsc.subcore_barrier()

                if edge_split:
                    base_b = cid * (blocks_total // 2) + sid * bpt
                else:
                    base_b = sid * bpt

                def ld_ed(blk_i, buf):
                    pltpu.sync_copy(ed_r.at[pl.ds((base_b + blk_i) * br, br)],
                                    ed_v.at[buf])
                    if weighted:
                        pltpu.sync_copy(
                            w_r.at[pl.ds((base_b + blk_i) * br, br)],
                            w_v.at[buf])

                def g_issue(buf):
                    for j in range(br):
                        pltpu.async_copy(x_r.at[ed_v.at[buf, j, 1]],
                                         rows_v.at[buf, pl.ds(j * 128, 128)],
                                         sem_g)

                def g_wait(buf):
                    for j in range(br):
                        pltpu.make_async_copy(
                            x_r.at[ed_v.at[buf, j, 1]],
                            rows_v.at[buf, pl.ds(j * 128, 128)], sem_g).wait()

                def s_issue(buf):
                    for j in range(br):
                        pltpu.async_copy(rows_v.at[buf, pl.ds(j * 128, 128)],
                                         acc.at[ed_v.at[buf, j, 0]], sem_s,
                                         add=True)

                def s_wait(buf):
                    for j in range(br):
                        pltpu.make_async_copy(
                            rows_v.at[buf, pl.ds(j * 128, 128)],
                            acc.at[ed_v.at[buf, j, 0]], sem_s).wait()

                def emul(buf):
                    for j in range(br):
                        def eg(g2, c_):
                            w16 = w_v[buf, j, pl.ds(g2 * 16, 16)]
                            for k in range(16):
                                wv = w16.at[jnp.full((16,), k, jnp.int32)].get(
                                    mode="promise_in_bounds")
                                e = j * 128 + g2 * 16 + k
                                for c2 in range(cw // 16):
                                    sl = pl.ds(c2 * 16, 16)
                                    rows_v[buf, e, sl] = rows_v[buf, e, sl] * wv
                            return c_
                        lax.fori_loop(0, 8, eg, 0)

                ld_ed(0, 0)
                g_issue(0)

                def outer(o, c):
                    for sub in range(2):
                        b = o * 2 + sub
                        nxt = 1 - sub
                        g_wait(sub)
                        pl.when(b >= 1)(lambda: s_wait(nxt))

                        def prefetch(b=b, nxt=nxt):
                            ld_ed(b + 1, nxt)
                            g_issue(nxt)
                        pl.when(b + 1 < bpt)(prefetch)
                        if weighted:
                            emul(sub)
                        s_issue(sub)
                    return c
                lax.fori_loop(0, bpt // 2, outer, 0)
                s_wait(1)
                plsc.subcore_barrier()

                def fl(j, c):
                    pltpu.sync_copy(acc.at[pl.ds(r0 + j * zr, zr)],
                                    out_r.at[pl.ds(r0 + j * zr, zr)])
                    return c
                lax.fori_loop(0, rpt // zr, fl, 0)
                plsc.subcore_barrier()

            pl.when(cid == core)(chunk_body)

    f = pl.kernel(body, out_type=out_type, mesh=mesh, scratch_types=scratch,
                  compiler_params=pltpu.CompilerParams(use_tc_tiling_on_sc=False))
    eargs = [ed, w2] if weighted else [ed]
    res = f(*(list(xs) + eargs))
    return list(res) if isinstance(res, (list, tuple)) else [res]


def _pack_edges(dst, src, w, e_pad, pad_dst):
    e = dst.shape[0]
    dst = jnp.concatenate([dst, jnp.full((e_pad - e,), pad_dst, jnp.int32)])
    src = jnp.concatenate([src, jnp.zeros((e_pad - e,), jnp.int32)])
    ed = jnp.concatenate([dst.reshape(-1, 1, 128), src.reshape(-1, 1, 128)],
                         axis=1)
    if w is None:
        return ed
    w = jnp.concatenate([w, jnp.zeros((e_pad - e,), jnp.float32)])
    return ed, w.reshape(-1, 128)


def _mm_body(x_ref, w_ref, o_ref):
    o_ref[...] = jnp.dot(x_ref[...], w_ref[...], preferred_element_type=jnp.float32)


def _matmul(x, w, bm=1000):
    M, K = x.shape
    N = w.shape[1]
    return pl.pallas_call(
        _mm_body,
        grid=(M // bm,),
        in_specs=[
            pl.BlockSpec((bm, K), lambda i: (i, 0)),
            pl.BlockSpec((K, N), lambda i: (0, 0)),
        ],
        out_specs=pl.BlockSpec((bm, N), lambda i: (i, 0)),
        out_shape=jax.ShapeDtypeStruct((M, N), jnp.float32),
    )(x, w)


def _l2norm(x, eps=1e-12):
    n = jnp.linalg.norm(x, axis=1, keepdims=True)
    return x / jnp.maximum(n, eps)


def _gumbel_softmax(logits, key, tau):
    g = jax.random.gumbel(key, logits.shape, logits.dtype)
    return jax.nn.softmax((logits + g) / tau, axis=1)


def kernel(gu, gi, feat_visual, feat_textual, trs_visual, trs_textual, hyper_visual, hyper_textual, r_index, r_values, adj_index, adj_weight, num_inters_inv):
    gkey = jax.random.key(7)

    ihf = []
    for f, t, h in [(feat_visual, trs_visual, hyper_visual),
                    (feat_textual, trs_textual, hyper_textual)]:
        w = jnp.concatenate([t, h], axis=1)
        w = jnp.pad(w, ((0, 0), (0, 128 - w.shape[1])))
        ihf.append(_matmul(f, w))
    item_feats = [ihf[0][:, :EMBED_K], ihf[1][:, :EMBED_K]]
    ih_raw = [ihf[0][:, EMBED_K:EMBED_K + HYPER_NUM], ihf[1][:, EMBED_K:EMBED_K + HYPER_NUM]]

    red = _pack_edges(r_index[0], r_index[1], None, R_EPAD, R_ROWS - 1)

    ih2 = jnp.concatenate([ih_raw[0], ih_raw[1],
                           jnp.zeros((NUM_ITEMS, 8), jnp.float32)], axis=1)
    uh_parts = _sc_spmm([ih2], red, n_rows_pad=R_ROWS, weighted=False,
                        chunk_core=[0, 1], x_slot=[0, 0], cw=16, zr=112,
                        b_edges=256, edge_split=True)
    uh_raw = (uh_parts[0] + uh_parts[1])[:NUM_USERS]

    uf_raw = _sc_spmm(item_feats, red, n_rows_pad=R_ROWS, weighted=False,
                      chunk_core=[0, 1], x_slot=[0, 1], cw=64, zr=112,
                      b_edges=128)

    i_hyper, u_hyper = [], []
    for m in range(2):
        i_hyper.append(_gumbel_softmax(ih_raw[m], jax.random.fold_in(gkey, 2 * m), TAU))
        u_hyper.append(_gumbel_softmax(uh_raw[:, 4 * m:4 * m + 4],
                                       jax.random.fold_in(gkey, 2 * m + 1), TAU))

    user_feats = [uf_raw[m][:NUM_USERS] * num_inters_inv[:NUM_USERS, None]
                  for m in range(2)]

    ego = jnp.concatenate([gu, gi], axis=0)
    mge_in = [jnp.concatenate([user_feats[m], item_feats[m]], axis=0) for m in range(2)]

    aed = _pack_edges(adj_index[0], adj_index[1], adj_weight, A_EPAD, 0)
    x192 = [ego[:, :32], ego[:, 32:],
            mge_in[0][:, :32], mge_in[0][:, 32:],
            mge_in[1][:, :32], mge_in[1][:, 32:]]
    y = _sc_spmm(x192, aed, n_rows_pad=A_ROWS, weighted=True,
                 chunk_core=[0, 1, 0, 1, 0, 1], x_slot=[0, 1, 2, 3, 4, 5],
                 cw=32, zr=112, b_edges=256)
    y = [v[:N_NODES] for v in y]
    layer1 = jnp.concatenate([y[0], y[1]], axis=1)
    mge_out = [jnp.concatenate([y[2], y[3]], axis=1),
               jnp.concatenate([y[4], y[5]], axis=1)]
    l2 = _sc_spmm([y[0], y[1]], aed, n_rows_pad=A_ROWS, weighted=True,
                  chunk_core=[0, 1], x_slot=[0, 1], cw=32, zr=112,
                  b_edges=256)
    layer2 = jnp.concatenate([v[:N_NODES] for v in l2], axis=1)

    cge = (ego + layer1 + layer2) / 3.0
    lge = cge + _l2norm(mge_out[0]) + _l2norm(mge_out[1])

    item_cge = cge[NUM_USERS:]
    hyper_outs = []
    ghe = 0.0
    for m in range(2):
        lat = i_hyper[m].T @ item_cge
        i_ret = i_hyper[m] @ lat
        u_ret = u_hyper[m] @ lat
        hyper_outs += [u_ret, i_ret]
        ghe = ghe + jnp.concatenate([u_ret, i_ret], axis=0)
    all_embs = lge + ALPHA * _l2norm(ghe)
    return (all_embs[:NUM_USERS], all_embs[NUM_USERS:],
            hyper_outs[0], hyper_outs[1], hyper_outs[2], hyper_outs[3])
